# baseline (speedup 1.0000x reference)
import functools

import jax
import jax.numpy as jnp
from jax import lax
from jax.experimental import pallas as pl
from jax.experimental.pallas import tpu as pltpu
from jax.experimental.pallas import tpu_sc as plsc

N_PTS = 524288
D = 8
N_GRID = 1024
NC = 2
NS = 16
L = 16
NW = NC * NS
PER_W = N_PTS // NW
BLK = 2048
NBLK = PER_W // BLK
GRPS = BLK // L

_MAGIC = 0x5F3759DF


def _interp_body(xt_hbm, params_hbm, grid_hbm, knn_hbm, slopes_hbm, out_hbm,
                 xv0, xv1, ov0, ov1, gv, kv, sv, av, bv, pv, isem, osem):
    wid = lax.axis_index("s") * NC + lax.axis_index("c")
    base_pt = wid * PER_W
    xvs, ovs, isems, osems = (xv0, xv1), (ov0, ov1), isem, osem

    pltpu.async_copy(xt_hbm.at[:, pl.ds(base_pt, BLK)], xv0, isems[0])

    pltpu.sync_copy(grid_hbm, gv)
    pltpu.sync_copy(knn_hbm, kv)
    pltpu.sync_copy(slopes_hbm, sv)
    pltpu.sync_copy(params_hbm, pv.at[pl.ds(0, 2)])

    one_v = jnp.full((L,), 1.0, jnp.float32)
    pvec = pv[pl.ds(0, L)]
    sig2 = pvec[0]
    ell = pvec[1]
    ell2 = ell * ell
    inv_ell_v = (one_v * ell) / (one_v * ell2)
    scale = ell2 * sig2
    bscale = ell * sig2
    gvec = gv[pl.ds(0, L)]
    h = gvec[1] - gvec[0]
    inv_h = (one_v / (one_v * h)) * inv_ell_v

    @plsc.parallel_loop(0, N_GRID // L, unroll=4)
    def tab_body(i):
        c = i * L
        gvi = gv[pl.ds(c, L)]
        kvi = kv[pl.ds(c, L)]
        svi = sv[pl.ds(c, L)]
        av[pl.ds(c, L)] = (kvi - svi * gvi) * scale
        bv[pl.ds(c, L)] = svi * bscale

    half = jnp.full((L,), 0.5, jnp.float32)
    three_half = jnp.full((L,), 1.5, jnp.float32)
    tiny = jnp.full((L,), 1e-30, jnp.float32)
    last = jnp.full((L,), N_GRID - 1, jnp.int32)
    zero_i = jnp.full((L,), 0, jnp.int32)
    zero_f = jnp.full((L,), 0.0, jnp.float32)
    magic = jnp.full((L,), _MAGIC, jnp.int32)

    def do_block(xv, ov):
        @plsc.parallel_loop(0, GRPS, unroll=16)
        def grp_body(g):
            c = g * L
            s = zero_f
            for j in range(D):
                col = xv[j, pl.ds(c, L)]
                s = s + col * col
            ss = jnp.maximum(s, tiny)
            bits = lax.bitcast_convert_type(ss, jnp.int32)
            y = lax.bitcast_convert_type(magic - (bits >> 1), jnp.float32)
            hs = half * ss
            y = y * (three_half - hs * y * y)
            y = y * (three_half - hs * y * y)
            d = ss * y
            k = (d * inv_h).astype(jnp.int32)
            k = jnp.minimum(k, last)
            k = jnp.where(s > 0.0, k, last)
            a = plsc.load_gather(av, [k])
            b2 = plsc.load_gather(bv, [k])
            ov[pl.ds(c, L)] = a + b2 * d

    for b in range(NBLK):
        cur, nxt = b % 2, (b + 1) % 2
        if b + 1 < NBLK:
            pltpu.async_copy(
                xt_hbm.at[:, pl.ds(base_pt + (b + 1) * BLK, BLK)],
                xvs[nxt], isems[nxt])
        pltpu.make_async_copy(
            xt_hbm.at[:, pl.ds(base_pt + b * BLK, BLK)],
            xvs[cur], isems[cur]).wait()
        if b >= 2:
            pltpu.make_async_copy(
                ovs[cur], out_hbm.at[pl.ds(base_pt + (b - 2) * BLK, BLK)],
                osems[cur]).wait()
        do_block(xvs[cur], ovs[cur])
        pltpu.async_copy(
            ovs[cur], out_hbm.at[pl.ds(base_pt + b * BLK, BLK)], osems[cur])
    for b in (NBLK - 2, NBLK - 1):
        cur = b % 2
        pltpu.make_async_copy(
            ovs[cur], out_hbm.at[pl.ds(base_pt + b * BLK, BLK)],
            osems[cur]).wait()


@jax.jit
def _interp(xt, params, grid, knn, slopes):
    call = pl.kernel(
        _interp_body,
        out_type=jax.ShapeDtypeStruct((N_PTS,), jnp.float32),
        mesh=plsc.VectorSubcoreMesh(core_axis_name="c", subcore_axis_name="s",
                                    num_cores=NC, num_subcores=NS),
        scratch_types=[
            pltpu.VMEM((D, BLK), jnp.float32),
            pltpu.VMEM((D, BLK), jnp.float32),
            pltpu.VMEM((BLK,), jnp.float32),
            pltpu.VMEM((BLK,), jnp.float32),
            pltpu.VMEM((N_GRID,), jnp.float32),
            pltpu.VMEM((N_GRID,), jnp.float32),
            pltpu.VMEM((N_GRID,), jnp.float32),
            pltpu.VMEM((N_GRID,), jnp.float32),
            pltpu.VMEM((N_GRID,), jnp.float32),
            pltpu.VMEM((L,), jnp.float32),
            [pltpu.SemaphoreType.DMA, pltpu.SemaphoreType.DMA],
            [pltpu.SemaphoreType.DMA, pltpu.SemaphoreType.DMA],
        ],
        compiler_params=pltpu.CompilerParams(needs_layout_passes=False),
    )
    return call(xt, params, grid, knn, slopes)


def kernel(x, params, distance_grid, knn, slopes):
    return _interp(x.T, params, distance_grid, knn, slopes)


# speedup vs baseline: 1.5575x; 1.5575x over previous
import functools

import jax
import jax.numpy as jnp
from jax import lax
from jax.experimental import pallas as pl
from jax.experimental.pallas import tpu as pltpu
from jax.experimental.pallas import tpu_sc as plsc

N_PTS = 524288
D = 8
N_GRID = 1024
NC = 2
NS = 16
L = 16
NW = NC * NS
PER_W = N_PTS // NW
BLK = 4096
NBLK = PER_W // BLK
GRPS = BLK // L

_MAGIC = 0x5F3759DF


def _interp_body(xt_hbm, params_hbm, grid_hbm, knn_hbm, slopes_hbm, out_hbm,
                 xv0, xv1, ov0, ov1, gv, kv, sv, av, bv, pv, isem, osem):
    wid = lax.axis_index("s") * NC + lax.axis_index("c")
    base_pt = wid * PER_W
    xvs, ovs, isems, osems = (xv0, xv1), (ov0, ov1), isem, osem

    pltpu.async_copy(xt_hbm.at[:, pl.ds(base_pt, BLK)], xv0, isems[0])

    pltpu.sync_copy(grid_hbm, gv)
    pltpu.sync_copy(knn_hbm, kv)
    pltpu.sync_copy(slopes_hbm, sv)
    pltpu.sync_copy(params_hbm, pv.at[pl.ds(0, 2)])

    one_v = jnp.full((L,), 1.0, jnp.float32)
    pvec = pv[pl.ds(0, L)]
    sig2 = pvec[0]
    ell = pvec[1]
    ell2 = ell * ell
    inv_ell_v = (one_v * ell) / (one_v * ell2)
    scale = ell2 * sig2
    bscale = ell * sig2
    gvec = gv[pl.ds(0, L)]
    h = gvec[1] - gvec[0]
    inv_h = (one_v / (one_v * h)) * inv_ell_v

    @plsc.parallel_loop(0, N_GRID // L, unroll=4)
    def tab_body(i):
        c = i * L
        gvi = gv[pl.ds(c, L)]
        kvi = kv[pl.ds(c, L)]
        svi = sv[pl.ds(c, L)]
        av[pl.ds(c, L)] = (kvi - svi * gvi) * scale
        bv[pl.ds(c, L)] = svi * bscale

    half = jnp.full((L,), 0.5, jnp.float32)
    three_half = jnp.full((L,), 1.5, jnp.float32)
    tiny = jnp.full((L,), 1e-30, jnp.float32)
    last = jnp.full((L,), N_GRID - 1, jnp.int32)
    zero_i = jnp.full((L,), 0, jnp.int32)
    zero_f = jnp.full((L,), 0.0, jnp.float32)
    magic = jnp.full((L,), _MAGIC, jnp.int32)

    def do_block(xv, ov):
        @plsc.parallel_loop(0, GRPS, unroll=8)
        def grp_body(g):
            c = g * L
            s = zero_f
            for j in range(D):
                col = xv[j, pl.ds(c, L)]
                s = s + col * col
            ss = jnp.maximum(s, tiny)
            bits = lax.bitcast_convert_type(ss, jnp.int32)
            y = lax.bitcast_convert_type(magic - (bits >> 1), jnp.float32)
            hs = half * ss
            y = y * (three_half - hs * y * y)
            y = y * (three_half - hs * y * y)
            d = ss * y
            k = (d * inv_h).astype(jnp.int32)
            k = jnp.minimum(k, last)
            k = jnp.where(s > 0.0, k, last)
            a = plsc.load_gather(av, [k])
            b2 = plsc.load_gather(bv, [k])
            ov[pl.ds(c, L)] = a + b2 * d

    for b in range(NBLK):
        cur, nxt = b % 2, (b + 1) % 2
        if b + 1 < NBLK:
            pltpu.async_copy(
                xt_hbm.at[:, pl.ds(base_pt + (b + 1) * BLK, BLK)],
                xvs[nxt], isems[nxt])
        pltpu.make_async_copy(
            xt_hbm.at[:, pl.ds(base_pt + b * BLK, BLK)],
            xvs[cur], isems[cur]).wait()
        if b >= 2:
            pltpu.make_async_copy(
                ovs[cur], out_hbm.at[pl.ds(base_pt + (b - 2) * BLK, BLK)],
                osems[cur]).wait()
        do_block(xvs[cur], ovs[cur])
        pltpu.async_copy(
            ovs[cur], out_hbm.at[pl.ds(base_pt + b * BLK, BLK)], osems[cur])
    for b in (NBLK - 2, NBLK - 1):
        cur = b % 2
        pltpu.make_async_copy(
            ovs[cur], out_hbm.at[pl.ds(base_pt + b * BLK, BLK)],
            osems[cur]).wait()


@jax.jit
def _interp(xt, params, grid, knn, slopes):
    call = pl.kernel(
        _interp_body,
        out_type=jax.ShapeDtypeStruct((N_PTS,), jnp.float32),
        mesh=plsc.VectorSubcoreMesh(core_axis_name="c", subcore_axis_name="s",
                                    num_cores=NC, num_subcores=NS),
        scratch_types=[
            pltpu.VMEM((D, BLK), jnp.float32),
            pltpu.VMEM((D, BLK), jnp.float32),
            pltpu.VMEM((BLK,), jnp.float32),
            pltpu.VMEM((BLK,), jnp.float32),
            pltpu.VMEM((N_GRID,), jnp.float32),
            pltpu.VMEM((N_GRID,), jnp.float32),
            pltpu.VMEM((N_GRID,), jnp.float32),
            pltpu.VMEM((N_GRID,), jnp.float32),
            pltpu.VMEM((N_GRID,), jnp.float32),
            pltpu.VMEM((L,), jnp.float32),
            [pltpu.SemaphoreType.DMA, pltpu.SemaphoreType.DMA],
            [pltpu.SemaphoreType.DMA, pltpu.SemaphoreType.DMA],
        ],
        compiler_params=pltpu.CompilerParams(needs_layout_passes=False),
    )
    return call(xt, params, grid, knn, slopes)


def kernel(x, params, distance_grid, knn, slopes):
    return _interp(x.T, params, distance_grid, knn, slopes)
